# baseline (speedup 1.0000x reference)
import functools

import jax
import jax.numpy as jnp
from jax import lax
from jax.experimental import pallas as pl
from jax.experimental.pallas import tpu as pltpu
from jax.experimental.pallas import tpu_sc as plsc

_B = 4096
_L = 64
_D = 64
_NPAIR = _B * _L
_NW = 32
_PER_W = _NPAIR // _NW
_C = 128
_CHUNKS = _PER_W // _C


def _sc_gather_body(uid_hbm, iid_hbm, mfu_t, mfi_t, mlpu_t, mlpi_t,
                    out_mfu, out_mfi, out_mlpu, out_mlpi,
                    idx_u, idx_i, buf_mfu, buf_mfi, buf_mlpu, buf_mlpi, sem):
    c = lax.axis_index("c")
    s = lax.axis_index("s")
    wid = s * 2 + c
    wbase = wid * _PER_W

    def chunk(k, carry):
        base = pl.multiple_of(wbase + k * _C, _C)
        pltpu.sync_copy(uid_hbm.at[pl.ds(base, _C)], idx_u)
        pltpu.sync_copy(iid_hbm.at[pl.ds(base, _C)], idx_i)
        cp0 = pltpu.async_copy(mfu_t.at[idx_u], buf_mfu, sem)
        cp1 = pltpu.async_copy(mfi_t.at[idx_i], buf_mfi, sem)
        cp2 = pltpu.async_copy(mlpu_t.at[idx_u], buf_mlpu, sem)
        cp3 = pltpu.async_copy(mlpi_t.at[idx_i], buf_mlpi, sem)
        cp0.wait()
        cp1.wait()
        cp2.wait()
        cp3.wait()
        pltpu.sync_copy(buf_mfu, out_mfu.at[pl.ds(base, _C)])
        pltpu.sync_copy(buf_mfi, out_mfi.at[pl.ds(base, _C)])
        pltpu.sync_copy(buf_mlpu, out_mlpu.at[pl.ds(base, _C)])
        pltpu.sync_copy(buf_mlpi, out_mlpi.at[pl.ds(base, _C)])
        return carry

    lax.fori_loop(0, _CHUNKS, chunk, 0)


@jax.jit
def _sc_gather(uid, iid, mfu_t, mfi_t, mlpu_t, mlpi_t):
    mesh = plsc.VectorSubcoreMesh(core_axis_name="c", subcore_axis_name="s")
    row = jax.ShapeDtypeStruct((_NPAIR, _D), jnp.float32)
    fn = pl.kernel(
        _sc_gather_body,
        out_type=[row, row, row, row],
        mesh=mesh,
        compiler_params=pltpu.CompilerParams(use_tc_tiling_on_sc=False),
        scratch_types=[
            pltpu.VMEM((_C,), jnp.int32),
            pltpu.VMEM((_C,), jnp.int32),
            pltpu.VMEM((_C, _D), jnp.float32),
            pltpu.VMEM((_C, _D), jnp.float32),
            pltpu.VMEM((_C, _D), jnp.float32),
            pltpu.VMEM((_C, _D), jnp.float32),
            pltpu.SemaphoreType.DMA,
        ],
    )
    return fn(uid, iid, mfu_t, mfi_t, mlpu_t, mlpi_t)


_R = 4096
_BB = _R // _L


def _tc_body(mfu_ref, mfi_ref, mlpu_ref, mlpi_ref, w1t_ref, w1b_ref,
             b1_ref, gw_ref, mw_ref, sgt_ref, smt_ref, out_ref):
    prod = mfu_ref[...] * mfi_ref[...]
    gvec = jnp.dot(prod, gw_ref[...], preferred_element_type=jnp.float32)
    h = jnp.maximum(
        jnp.dot(mlpu_ref[...], w1t_ref[...], preferred_element_type=jnp.float32)
        + jnp.dot(mlpi_ref[...], w1b_ref[...], preferred_element_type=jnp.float32)
        + b1_ref[...], 0.0)
    mvec = jnp.dot(h, mw_ref[...], preferred_element_type=jnp.float32)
    out_ref[...] = (
        jnp.dot(sgt_ref[...], gvec, preferred_element_type=jnp.float32)
        + jnp.dot(smt_ref[...], mvec, preferred_element_type=jnp.float32))


@jax.jit
def _tc_dense(mfu_g, mfi_g, mlpu_g, mlpi_g, w1t, w1b, b1r, gw, mw, sgt, smt):
    n_blocks = _NPAIR // _R
    row_spec = pl.BlockSpec((_R, _D), lambda i: (i, 0))
    full = lambda shape: pl.BlockSpec(shape, lambda i: (0, 0))
    return pl.pallas_call(
        _tc_body,
        grid=(n_blocks,),
        in_specs=[
            row_spec, row_spec, row_spec, row_spec,
            full((_D, _D)), full((_D, _D)), full((1, _D)),
            full((_D, 1)), full((_D, 1)),
            full((_BB, _R)), full((_BB, _R)),
        ],
        out_specs=pl.BlockSpec((_BB, 1), lambda i: (i, 0)),
        out_shape=jax.ShapeDtypeStruct((_B, 1), jnp.float32),
    )(mfu_g, mfi_g, mlpu_g, mlpi_g, w1t, w1b, b1r, gw, mw, sgt, smt)


def kernel(user_id, item_id, mf_user_emb, mf_item_emb, gmf_w, gmf_b,
           mlp_user_emb, mlp_item_emb, mlp_w1, mlp_b1, mlp_w, mlp_b,
           fin_w, fin_b):
    uid = user_id.reshape(-1).astype(jnp.int32)
    iid = item_id.reshape(-1).astype(jnp.int32)
    mfu_g, mfi_g, mlpu_g, mlpi_g = _sc_gather(
        uid, iid, mf_user_emb, mf_item_emb, mlp_user_emb, mlp_item_emb)

    wg = fin_w[:_L, 0]
    wm = fin_w[_L:, 0]
    eye = jnp.eye(_BB, dtype=jnp.float32)
    sgt = jnp.kron(eye, wg[None, :])
    smt = jnp.kron(eye, wm[None, :])
    w1t = mlp_w1[:_D]
    w1b = mlp_w1[_D:]
    b1r = mlp_b1.reshape(1, _D)

    pred = _tc_dense(mfu_g, mfi_g, mlpu_g, mlpi_g, w1t, w1b, b1r,
                     gmf_w, mlp_w, sgt, smt)
    cst = gmf_b[0] * jnp.sum(wg) + mlp_b[0] * jnp.sum(wm) + fin_b[0]
    return pred.reshape(_B) + cst


# speedup vs baseline: 1.0721x; 1.0721x over previous
import functools

import jax
import jax.numpy as jnp
from jax import lax
from jax.experimental import pallas as pl
from jax.experimental.pallas import tpu as pltpu
from jax.experimental.pallas import tpu_sc as plsc

_B = 4096
_L = 64
_D = 64
_NPAIR = _B * _L
_NW = 32
_PER_W = _NPAIR // _NW
_C = 128
_CHUNKS = _PER_W // _C
_NLANE = 16


def _sc_gather_body(uid_hbm, iid_hbm, mfu_t, mfi_t, mlpu_t, mlpi_t,
                    out_prod, out_mlpu, out_mlpi,
                    idxu_all, idxi_all, bufs, gsem, wsem):
    c = lax.axis_index("c")
    s = lax.axis_index("s")
    wid = s * 2 + c
    wbase = wid * _PER_W

    pltpu.sync_copy(uid_hbm.at[pl.ds(wbase, _PER_W)], idxu_all)
    pltpu.sync_copy(iid_hbm.at[pl.ds(wbase, _PER_W)], idxi_all)

    def idx_slices(k):
        off = pl.multiple_of(k * _C, _C)
        return idxu_all.at[pl.ds(off, _C)], idxi_all.at[pl.ds(off, _C)]

    def fire_gathers(k, bset):
        iu, ii = idx_slices(k)
        mfu_b, mfi_b, mlpu_b, mlpi_b = bset
        pltpu.async_copy(mfu_t.at[iu], mfu_b, gsem)
        pltpu.async_copy(mfi_t.at[ii], mfi_b, gsem)
        pltpu.async_copy(mlpu_t.at[iu], mlpu_b, gsem)
        pltpu.async_copy(mlpi_t.at[ii], mlpi_b, gsem)

    def wait_gathers(k, bset):
        iu, ii = idx_slices(k)
        mfu_b, mfi_b, mlpu_b, mlpi_b = bset
        pltpu.make_async_copy(mfu_t.at[iu], mfu_b, gsem).wait()
        pltpu.make_async_copy(mfi_t.at[ii], mfi_b, gsem).wait()
        pltpu.make_async_copy(mlpu_t.at[iu], mlpu_b, gsem).wait()
        pltpu.make_async_copy(mlpi_t.at[ii], mlpi_b, gsem).wait()

    def out_slices(k):
        base = pl.multiple_of(wbase + k * _C, _C)
        return (out_prod.at[pl.ds(base, _C)],
                out_mlpu.at[pl.ds(base, _C)],
                out_mlpi.at[pl.ds(base, _C)])

    def compute_product(bset):
        mfu_b, mfi_b = bset[0], bset[1]

        def row(r, carry):
            for cc in range(_D // _NLANE):
                sl = pl.ds(cc * _NLANE, _NLANE)
                mfu_b[r, sl] = mfu_b[r, sl] * mfi_b[r, sl]
            return carry

        lax.fori_loop(0, _C, row, 0, unroll=2)

    def fire_writes(k, bset):
        op, ou, oi = out_slices(k)
        pltpu.async_copy(bset[0], op, wsem)
        pltpu.async_copy(bset[2], ou, wsem)
        pltpu.async_copy(bset[3], oi, wsem)

    def wait_writes(k, bset):
        op, ou, oi = out_slices(k)
        pltpu.make_async_copy(bset[0], op, wsem).wait()
        pltpu.make_async_copy(bset[2], ou, wsem).wait()
        pltpu.make_async_copy(bset[3], oi, wsem).wait()

    fire_gathers(0, bufs[0])

    def outer(t, carry):
        for b in range(2):
            k = t * 2 + b
            cur = bufs[b]
            other = bufs[1 - b]

            @pl.when(k >= 1)
            def _():
                wait_writes(k - 1, other)

            @pl.when(k + 1 < _CHUNKS)
            def _():
                fire_gathers(k + 1, other)

            wait_gathers(k, cur)
            compute_product(cur)
            fire_writes(k, cur)
        return carry

    lax.fori_loop(0, _CHUNKS // 2, outer, 0)
    wait_writes(_CHUNKS - 1, bufs[(_CHUNKS - 1) % 2])


@jax.jit
def _sc_gather(uid, iid, mfu_t, mfi_t, mlpu_t, mlpi_t):
    mesh = plsc.VectorSubcoreMesh(core_axis_name="c", subcore_axis_name="s")
    row = jax.ShapeDtypeStruct((_NPAIR, _D), jnp.float32)
    buf = pltpu.VMEM((_C, _D), jnp.float32)
    fn = pl.kernel(
        _sc_gather_body,
        out_type=[row, row, row],
        mesh=mesh,
        compiler_params=pltpu.CompilerParams(use_tc_tiling_on_sc=False),
        scratch_types=[
            pltpu.VMEM((_PER_W,), jnp.int32),
            pltpu.VMEM((_PER_W,), jnp.int32),
            ((buf, buf, buf, buf), (buf, buf, buf, buf)),
            pltpu.SemaphoreType.DMA,
            pltpu.SemaphoreType.DMA,
        ],
    )
    return fn(uid, iid, mfu_t, mfi_t, mlpu_t, mlpi_t)


_R = 4096
_BB = _R // _L


def _tc_body(prod_ref, mlpu_ref, mlpi_ref, w1t_ref, w1b_ref,
             b1_ref, gw_ref, mw_ref, sgt_ref, smt_ref, out_ref):
    gvec = jnp.dot(prod_ref[...], gw_ref[...],
                   preferred_element_type=jnp.float32)
    h = jnp.maximum(
        jnp.dot(mlpu_ref[...], w1t_ref[...], preferred_element_type=jnp.float32)
        + jnp.dot(mlpi_ref[...], w1b_ref[...], preferred_element_type=jnp.float32)
        + b1_ref[...], 0.0)
    mvec = jnp.dot(h, mw_ref[...], preferred_element_type=jnp.float32)
    out_ref[...] = (
        jnp.dot(sgt_ref[...], gvec, preferred_element_type=jnp.float32)
        + jnp.dot(smt_ref[...], mvec, preferred_element_type=jnp.float32))


@jax.jit
def _tc_dense(prod_g, mlpu_g, mlpi_g, w1t, w1b, b1r, gw, mw, sgt, smt):
    n_blocks = _NPAIR // _R
    row_spec = pl.BlockSpec((_R, _D), lambda i: (i, 0))
    full = lambda shape: pl.BlockSpec(shape, lambda i: (0, 0))
    return pl.pallas_call(
        _tc_body,
        grid=(n_blocks,),
        in_specs=[
            row_spec, row_spec, row_spec,
            full((_D, _D)), full((_D, _D)), full((1, _D)),
            full((_D, 1)), full((_D, 1)),
            full((_BB, _R)), full((_BB, _R)),
        ],
        out_specs=pl.BlockSpec((_BB, 1), lambda i: (i, 0)),
        out_shape=jax.ShapeDtypeStruct((_B, 1), jnp.float32),
    )(prod_g, mlpu_g, mlpi_g, w1t, w1b, b1r, gw, mw, sgt, smt)


def kernel(user_id, item_id, mf_user_emb, mf_item_emb, gmf_w, gmf_b,
           mlp_user_emb, mlp_item_emb, mlp_w1, mlp_b1, mlp_w, mlp_b,
           fin_w, fin_b):
    uid = user_id.reshape(-1).astype(jnp.int32)
    iid = item_id.reshape(-1).astype(jnp.int32)
    prod_g, mlpu_g, mlpi_g = _sc_gather(
        uid, iid, mf_user_emb, mf_item_emb, mlp_user_emb, mlp_item_emb)

    wg = fin_w[:_L, 0]
    wm = fin_w[_L:, 0]
    eye = jnp.eye(_BB, dtype=jnp.float32)
    sgt = jnp.kron(eye, wg[None, :])
    smt = jnp.kron(eye, wm[None, :])
    w1t = mlp_w1[:_D]
    w1b = mlp_w1[_D:]
    b1r = mlp_b1.reshape(1, _D)

    pred = _tc_dense(prod_g, mlpu_g, mlpi_g, w1t, w1b, b1r,
                     gmf_w, mlp_w, sgt, smt)
    cst = gmf_b[0] * jnp.sum(wg) + mlp_b[0] * jnp.sum(wm) + fin_b[0]
    return pred.reshape(_B) + cst


# speedup vs baseline: 1.2342x; 1.1512x over previous
import functools

import jax
import jax.numpy as jnp
from jax import lax
from jax.experimental import pallas as pl
from jax.experimental.pallas import tpu as pltpu
from jax.experimental.pallas import tpu_sc as plsc

_B = 4096
_L = 64
_D = 64
_NPAIR = _B * _L
_NW = 32
_PER_W = _NPAIR // _NW
_C = 128
_CHUNKS = _PER_W // _C
_NLANE = 16


def _sc_gather_body(uid_hbm, iid_hbm, mfu_t, mfi_t, mlpu_t, mlpi_t,
                    out_mf, out_mlp,
                    idxu_all, idxi_all, bufs, gsem, wsem):
    c = lax.axis_index("c")
    s = lax.axis_index("s")
    wid = s * 2 + c
    wbase = wid * _PER_W

    pltpu.sync_copy(uid_hbm.at[pl.ds(wbase, _PER_W)], idxu_all)
    pltpu.sync_copy(iid_hbm.at[pl.ds(wbase, _PER_W)], idxi_all)

    lo = pl.ds(0, _D)
    hi = pl.ds(_D, _D)

    def idx_slices(k):
        off = pl.multiple_of(k * _C, _C)
        return idxu_all.at[pl.ds(off, _C)], idxi_all.at[pl.ds(off, _C)]

    def fire_gathers(k, bset):
        iu, ii = idx_slices(k)
        mfu_b, mfi_b, mlpu_b, mlpi_b = bset
        pltpu.async_copy(mfu_t.at[iu], mfu_b, gsem)
        pltpu.async_copy(mfi_t.at[ii], mfi_b, gsem)
        pltpu.async_copy(mlpu_t.at[iu], mlpu_b, gsem)
        pltpu.async_copy(mlpi_t.at[ii], mlpi_b, gsem)

    def wait_gathers(k, bset):
        iu, ii = idx_slices(k)
        mfu_b, mfi_b, mlpu_b, mlpi_b = bset
        pltpu.make_async_copy(mfu_t.at[iu], mfu_b, gsem).wait()
        pltpu.make_async_copy(mfi_t.at[ii], mfi_b, gsem).wait()
        pltpu.make_async_copy(mlpu_t.at[iu], mlpu_b, gsem).wait()
        pltpu.make_async_copy(mlpi_t.at[ii], mlpi_b, gsem).wait()

    def out_slices(k):
        base = pl.ds(pl.multiple_of(wbase + k * _C, _C), _C)
        return (out_mf.at[base, lo], out_mf.at[base, hi],
                out_mlp.at[base, lo], out_mlp.at[base, hi])

    def fire_writes(k, bset):
        outs = out_slices(k)
        for b, o in zip(bset, outs):
            pltpu.async_copy(b, o, wsem)

    def wait_writes(k, bset):
        outs = out_slices(k)
        for b, o in zip(bset, outs):
            pltpu.make_async_copy(b, o, wsem).wait()

    fire_gathers(0, bufs[0])

    def outer(t, carry):
        for b in range(2):
            k = t * 2 + b
            cur = bufs[b]
            other = bufs[1 - b]

            @pl.when(k >= 1)
            def _():
                wait_writes(k - 1, other)

            @pl.when(k + 1 < _CHUNKS)
            def _():
                fire_gathers(k + 1, other)

            wait_gathers(k, cur)
            fire_writes(k, cur)
        return carry

    lax.fori_loop(0, _CHUNKS // 2, outer, 0)
    wait_writes(_CHUNKS - 1, bufs[(_CHUNKS - 1) % 2])


@jax.jit
def _sc_gather(uid, iid, mfu_t, mfi_t, mlpu_t, mlpi_t):
    mesh = plsc.VectorSubcoreMesh(core_axis_name="c", subcore_axis_name="s")
    cat = jax.ShapeDtypeStruct((_NPAIR, 2 * _D), jnp.float32)
    buf = pltpu.VMEM((_C, _D), jnp.float32)
    fn = pl.kernel(
        _sc_gather_body,
        out_type=[cat, cat],
        mesh=mesh,
        compiler_params=pltpu.CompilerParams(use_tc_tiling_on_sc=False),
        scratch_types=[
            pltpu.VMEM((_PER_W,), jnp.int32),
            pltpu.VMEM((_PER_W,), jnp.int32),
            ((buf, buf, buf, buf), (buf, buf, buf, buf)),
            pltpu.SemaphoreType.DMA,
            pltpu.SemaphoreType.DMA,
        ],
    )
    return fn(uid, iid, mfu_t, mfi_t, mlpu_t, mlpi_t)


_R = 4096
_BB = _R // _L


def _tc_body(mf_ref, mlp_ref, w1_ref, b1_ref, gw_ref, mw_ref,
             sgt_ref, smt_ref, out_ref):
    prod = mf_ref[:, :_D] * mf_ref[:, _D:]
    gvec = jnp.dot(prod, gw_ref[...], preferred_element_type=jnp.float32)
    h = jnp.maximum(
        jnp.dot(mlp_ref[...], w1_ref[...], preferred_element_type=jnp.float32)
        + b1_ref[...], 0.0)
    mvec = jnp.dot(h, mw_ref[...], preferred_element_type=jnp.float32)
    out_ref[...] = (
        jnp.dot(sgt_ref[...], gvec, preferred_element_type=jnp.float32)
        + jnp.dot(smt_ref[...], mvec, preferred_element_type=jnp.float32))


@jax.jit
def _tc_dense(mf_g, mlp_g, w1, b1r, gw, mw, sgt, smt):
    n_blocks = _NPAIR // _R
    row_spec = pl.BlockSpec((_R, 2 * _D), lambda i: (i, 0))
    full = lambda shape: pl.BlockSpec(shape, lambda i: (0, 0))
    return pl.pallas_call(
        _tc_body,
        grid=(n_blocks,),
        in_specs=[
            row_spec, row_spec,
            full((2 * _D, _D)), full((1, _D)),
            full((_D, 1)), full((_D, 1)),
            full((_BB, _R)), full((_BB, _R)),
        ],
        out_specs=pl.BlockSpec((_BB, 1), lambda i: (i, 0)),
        out_shape=jax.ShapeDtypeStruct((_B, 1), jnp.float32),
    )(mf_g, mlp_g, w1, b1r, gw, mw, sgt, smt)


def kernel(user_id, item_id, mf_user_emb, mf_item_emb, gmf_w, gmf_b,
           mlp_user_emb, mlp_item_emb, mlp_w1, mlp_b1, mlp_w, mlp_b,
           fin_w, fin_b):
    uid = user_id.reshape(-1).astype(jnp.int32)
    iid = item_id.reshape(-1).astype(jnp.int32)
    mf_g, mlp_g = _sc_gather(
        uid, iid, mf_user_emb, mf_item_emb, mlp_user_emb, mlp_item_emb)

    wg = fin_w[:_L, 0]
    wm = fin_w[_L:, 0]
    eye = jnp.eye(_BB, dtype=jnp.float32)
    sgt = jnp.kron(eye, wg[None, :])
    smt = jnp.kron(eye, wm[None, :])
    b1r = mlp_b1.reshape(1, _D)

    pred = _tc_dense(mf_g, mlp_g, mlp_w1, b1r, gmf_w, mlp_w, sgt, smt)
    cst = gmf_b[0] * jnp.sum(wg) + mlp_b[0] * jnp.sum(wm) + fin_b[0]
    return pred.reshape(_B) + cst


# speedup vs baseline: 1.4219x; 1.1521x over previous
import functools

import jax
import jax.numpy as jnp
from jax import lax
from jax.experimental import pallas as pl
from jax.experimental.pallas import tpu as pltpu
from jax.experimental.pallas import tpu_sc as plsc

_B = 4096
_L = 64
_D = 64
_V = 1000000
_NPAIR = _B * _L
_NW = 32
_PER_W = _NPAIR // _NW
_C = 128
_CHUNKS = _PER_W // _C
_VB = 1024


def _conv_body(pa_ref, pb_ref, out_ref):
    out_ref[:, :_D] = pa_ref[...].T
    out_ref[:, _D:] = pb_ref[...].T


@jax.jit
def _convert_pair(pa, pb):
    grid = (pl.cdiv(_V, _VB),)
    in_spec = pl.BlockSpec((_D, _VB), lambda j: (0, j))
    return pl.pallas_call(
        _conv_body,
        grid=grid,
        in_specs=[in_spec, in_spec],
        out_specs=pl.BlockSpec((_VB, 2 * _D), lambda j: (j, 0)),
        out_shape=jax.ShapeDtypeStruct((_V, 2 * _D), jnp.float32),
    )(pa, pb)


def _sc_gather_body(idx_hbm, tab_hbm, out_hbm, idx_all, bufs, gsem, wsem):
    c = lax.axis_index("c")
    s = lax.axis_index("s")
    wid = s * 2 + c
    wbase = wid * _PER_W

    pltpu.sync_copy(idx_hbm.at[pl.ds(wbase, _PER_W)], idx_all)

    def idx_slice(k):
        return idx_all.at[pl.ds(pl.multiple_of(k * _C, _C), _C)]

    def out_slice(k):
        return out_hbm.at[pl.ds(pl.multiple_of(wbase + k * _C, _C), _C)]

    def fire_gather(k, buf):
        pltpu.async_copy(tab_hbm.at[idx_slice(k)], buf, gsem)

    def wait_gather(k, buf):
        pltpu.make_async_copy(tab_hbm.at[idx_slice(k)], buf, gsem).wait()

    def fire_write(k, buf):
        pltpu.async_copy(buf, out_slice(k), wsem)

    def wait_write(k, buf):
        pltpu.make_async_copy(buf, out_slice(k), wsem).wait()

    fire_gather(0, bufs[0])

    def outer(t, carry):
        for b in range(2):
            k = t * 2 + b
            cur = bufs[b]
            other = bufs[1 - b]

            @pl.when(k >= 1)
            def _():
                wait_write(k - 1, other)

            @pl.when(k + 1 < _CHUNKS)
            def _():
                fire_gather(k + 1, other)

            wait_gather(k, cur)
            fire_write(k, cur)
        return carry

    lax.fori_loop(0, _CHUNKS // 2, outer, 0)
    wait_write(_CHUNKS - 1, bufs[(_CHUNKS - 1) % 2])


@jax.jit
def _sc_gather(idx, tab):
    mesh = plsc.VectorSubcoreMesh(core_axis_name="c", subcore_axis_name="s")
    buf = pltpu.VMEM((_C, 2 * _D), jnp.float32)
    fn = pl.kernel(
        _sc_gather_body,
        out_type=jax.ShapeDtypeStruct((_NPAIR, 2 * _D), jnp.float32),
        mesh=mesh,
        scratch_types=[
            pltpu.VMEM((_PER_W,), jnp.int32),
            (buf, buf),
            pltpu.SemaphoreType.DMA,
            pltpu.SemaphoreType.DMA,
        ],
    )
    return fn(idx, tab)


_R = 4096
_BB = _R // _L


def _tc_body(u_ref, i_ref, w1t_ref, w1b_ref, b1_ref, gw_ref, mw_ref,
             sgt_ref, smt_ref, out_ref):
    prod = u_ref[:, :_D] * i_ref[:, :_D]
    gvec = jnp.dot(prod, gw_ref[...], preferred_element_type=jnp.float32)
    h = jnp.maximum(
        jnp.dot(u_ref[:, _D:], w1t_ref[...], preferred_element_type=jnp.float32)
        + jnp.dot(i_ref[:, _D:], w1b_ref[...], preferred_element_type=jnp.float32)
        + b1_ref[...], 0.0)
    mvec = jnp.dot(h, mw_ref[...], preferred_element_type=jnp.float32)
    out_ref[...] = (
        jnp.dot(sgt_ref[...], gvec, preferred_element_type=jnp.float32)
        + jnp.dot(smt_ref[...], mvec, preferred_element_type=jnp.float32))


@jax.jit
def _tc_dense(u_g, i_g, w1t, w1b, b1r, gw, mw, sgt, smt):
    n_blocks = _NPAIR // _R
    row_spec = pl.BlockSpec((_R, 2 * _D), lambda i: (i, 0))
    full = lambda shape: pl.BlockSpec(shape, lambda i: (0, 0))
    return pl.pallas_call(
        _tc_body,
        grid=(n_blocks,),
        in_specs=[
            row_spec, row_spec,
            full((_D, _D)), full((_D, _D)), full((1, _D)),
            full((_D, 1)), full((_D, 1)),
            full((_BB, _R)), full((_BB, _R)),
        ],
        out_specs=pl.BlockSpec((_BB, 1), lambda i: (i, 0)),
        out_shape=jax.ShapeDtypeStruct((_B, 1), jnp.float32),
    )(u_g, i_g, w1t, w1b, b1r, gw, mw, sgt, smt)


def kernel(user_id, item_id, mf_user_emb, mf_item_emb, gmf_w, gmf_b,
           mlp_user_emb, mlp_item_emb, mlp_w1, mlp_b1, mlp_w, mlp_b,
           fin_w, fin_b):
    uid = user_id.reshape(-1).astype(jnp.int32)
    iid = item_id.reshape(-1).astype(jnp.int32)

    user_cat = _convert_pair(mf_user_emb.T, mlp_user_emb.T)
    u_g = _sc_gather(uid, user_cat)
    item_cat = _convert_pair(mf_item_emb.T, mlp_item_emb.T)
    i_g = _sc_gather(iid, item_cat)

    wg = fin_w[:_L, 0]
    wm = fin_w[_L:, 0]
    eye = jnp.eye(_BB, dtype=jnp.float32)
    sgt = jnp.kron(eye, wg[None, :])
    smt = jnp.kron(eye, wm[None, :])
    w1t = mlp_w1[:_D]
    w1b = mlp_w1[_D:]
    b1r = mlp_b1.reshape(1, _D)

    pred = _tc_dense(u_g, i_g, w1t, w1b, b1r, gmf_w, mlp_w, sgt, smt)
    cst = gmf_b[0] * jnp.sum(wg) + mlp_b[0] * jnp.sum(wm) + fin_b[0]
    return pred.reshape(_B) + cst


# speedup vs baseline: 2.5875x; 1.8198x over previous
import functools

import jax
import jax.numpy as jnp
from jax import lax
from jax.experimental import pallas as pl
from jax.experimental.pallas import tpu as pltpu
from jax.experimental.pallas import tpu_sc as plsc

_B = 4096
_L = 64
_D = 64
_V = 1000000
_NPAIR = _B * _L
_NW = 32
_PER_W = _NPAIR // _NW
_C = 128
_CHUNKS = _PER_W // _C
_VB = 16384


def _conv_body(pa_ref, pb_ref, eye_ref, out_ref):
    out_ref[:, :_D] = pa_ref[...].T
    out_ref[:, _D:] = pb_ref[...].T


@jax.jit
def _convert_pair(pa, pb, eye):
    grid = (pl.cdiv(_V, _VB),)
    in_spec = pl.BlockSpec((_D, _VB), lambda j: (0, j))
    return pl.pallas_call(
        _conv_body,
        grid=grid,
        in_specs=[in_spec, in_spec, pl.BlockSpec((_D, _D), lambda j: (0, 0))],
        out_specs=pl.BlockSpec((_VB, 2 * _D), lambda j: (j, 0)),
        out_shape=jax.ShapeDtypeStruct((_V, 2 * _D), jnp.float32),
    )(pa, pb, eye)


def _sc_gather_body(idx_hbm, tab_hbm, out_hbm, idx_all, bufs, gsem, wsem):
    c = lax.axis_index("c")
    s = lax.axis_index("s")
    wid = s * 2 + c
    wbase = wid * _PER_W

    pltpu.sync_copy(idx_hbm.at[pl.ds(wbase, _PER_W)], idx_all)

    def idx_slice(k):
        return idx_all.at[pl.ds(pl.multiple_of(k * _C, _C), _C)]

    def out_slice(k):
        return out_hbm.at[pl.ds(pl.multiple_of(wbase + k * _C, _C), _C)]

    def fire_gather(k, buf):
        pltpu.async_copy(tab_hbm.at[idx_slice(k)], buf, gsem)

    def wait_gather(k, buf):
        pltpu.make_async_copy(tab_hbm.at[idx_slice(k)], buf, gsem).wait()

    def fire_write(k, buf):
        pltpu.async_copy(buf, out_slice(k), wsem)

    def wait_write(k, buf):
        pltpu.make_async_copy(buf, out_slice(k), wsem).wait()

    fire_gather(0, bufs[0])

    def outer(t, carry):
        for b in range(2):
            k = t * 2 + b
            cur = bufs[b]
            other = bufs[1 - b]

            @pl.when(k >= 1)
            def _():
                wait_write(k - 1, other)

            @pl.when(k + 1 < _CHUNKS)
            def _():
                fire_gather(k + 1, other)

            wait_gather(k, cur)
            fire_write(k, cur)
        return carry

    lax.fori_loop(0, _CHUNKS // 2, outer, 0)
    wait_write(_CHUNKS - 1, bufs[(_CHUNKS - 1) % 2])


@jax.jit
def _sc_gather(idx, tab):
    mesh = plsc.VectorSubcoreMesh(core_axis_name="c", subcore_axis_name="s")
    buf = pltpu.VMEM((_C, 2 * _D), jnp.float32)
    fn = pl.kernel(
        _sc_gather_body,
        out_type=jax.ShapeDtypeStruct((_NPAIR, 2 * _D), jnp.float32),
        mesh=mesh,
        scratch_types=[
            pltpu.VMEM((_PER_W,), jnp.int32),
            (buf, buf),
            pltpu.SemaphoreType.DMA,
            pltpu.SemaphoreType.DMA,
        ],
    )
    return fn(idx, tab)


_R = 4096
_BB = _R // _L


def _tc_body(u_ref, i_ref, w1t_ref, w1b_ref, b1_ref, gw_ref, mw_ref,
             sgt_ref, smt_ref, out_ref):
    prod = u_ref[:, :_D] * i_ref[:, :_D]
    gvec = jnp.dot(prod, gw_ref[...], preferred_element_type=jnp.float32)
    h = jnp.maximum(
        jnp.dot(u_ref[:, _D:], w1t_ref[...], preferred_element_type=jnp.float32)
        + jnp.dot(i_ref[:, _D:], w1b_ref[...], preferred_element_type=jnp.float32)
        + b1_ref[...], 0.0)
    mvec = jnp.dot(h, mw_ref[...], preferred_element_type=jnp.float32)
    out_ref[...] = (
        jnp.dot(sgt_ref[...], gvec, preferred_element_type=jnp.float32)
        + jnp.dot(smt_ref[...], mvec, preferred_element_type=jnp.float32))


@jax.jit
def _tc_dense(u_g, i_g, w1t, w1b, b1r, gw, mw, sgt, smt):
    n_blocks = _NPAIR // _R
    row_spec = pl.BlockSpec((_R, 2 * _D), lambda i: (i, 0))
    full = lambda shape: pl.BlockSpec(shape, lambda i: (0, 0))
    return pl.pallas_call(
        _tc_body,
        grid=(n_blocks,),
        in_specs=[
            row_spec, row_spec,
            full((_D, _D)), full((_D, _D)), full((1, _D)),
            full((_D, 1)), full((_D, 1)),
            full((_BB, _R)), full((_BB, _R)),
        ],
        out_specs=pl.BlockSpec((_BB, 1), lambda i: (i, 0)),
        out_shape=jax.ShapeDtypeStruct((_B, 1), jnp.float32),
    )(u_g, i_g, w1t, w1b, b1r, gw, mw, sgt, smt)


def kernel(user_id, item_id, mf_user_emb, mf_item_emb, gmf_w, gmf_b,
           mlp_user_emb, mlp_item_emb, mlp_w1, mlp_b1, mlp_w, mlp_b,
           fin_w, fin_b):
    uid = user_id.reshape(-1).astype(jnp.int32)
    iid = item_id.reshape(-1).astype(jnp.int32)

    eye64 = jnp.eye(_D, dtype=jnp.float32)
    user_cat = _convert_pair(mf_user_emb.T, mlp_user_emb.T, eye64)
    u_g = _sc_gather(uid, user_cat)
    item_cat = _convert_pair(mf_item_emb.T, mlp_item_emb.T, eye64)
    i_g = _sc_gather(iid, item_cat)

    wg = fin_w[:_L, 0]
    wm = fin_w[_L:, 0]
    eye = jnp.eye(_BB, dtype=jnp.float32)
    sgt = jnp.kron(eye, wg[None, :])
    smt = jnp.kron(eye, wm[None, :])
    w1t = mlp_w1[:_D]
    w1b = mlp_w1[_D:]
    b1r = mlp_b1.reshape(1, _D)

    pred = _tc_dense(u_g, i_g, w1t, w1b, b1r, gmf_w, mlp_w, sgt, smt)
    cst = gmf_b[0] * jnp.sum(wg) + mlp_b[0] * jnp.sum(wm) + fin_b[0]
    return pred.reshape(_B) + cst
